# Initial kernel scaffold; baseline (speedup 1.0000x reference)
import functools

import jax
import jax.numpy as jnp
from jax import lax
from jax.experimental import pallas as pl
from jax.experimental.pallas import tpu as pltpu
from jax.experimental.pallas import tpu_sc as plsc

N = 10000
E = 320000
NC, NS = 2, 16
NW = NC * NS
LANES = 128
RT = 80
E_PAD = NW * RT * LANES
R_TOT = E_PAD // LANES
NPAD = 10112
RPT = NPAD // NS
DUMMY = N

_f32 = jnp.float32


def _mesh():
    return plsc.VectorSubcoreMesh(
        core_axis_name="c", subcore_axis_name="s",
        num_cores=NC, num_subcores=NS)


def _seg_sum_body(count, table, gidx, sidx, zrows, *refs):
    if count:
        (z16, ones16, acc_out, deg_out, odeg_out,
         gv, sv, ones_v, rows, acc_sh, deg_sh, odeg_sh,
         s0, s1, s2, s3) = refs
    else:
        (acc_out, gv, sv, rows, acc_sh, s0, s1, s2, s3) = refs
    sems = (s0, s1, s2, s3)
    c = lax.axis_index("c")
    s = lax.axis_index("s")
    w = c * NS + s

    pltpu.sync_copy(gidx.at[pl.ds(w * RT, RT)], gv)
    pltpu.sync_copy(sidx.at[pl.ds(w * RT, RT)], sv)
    pltpu.sync_copy(zrows, acc_sh.at[pl.ds(s * RPT, RPT)])
    if count:
        pltpu.sync_copy(ones16, ones_v)
        pltpu.sync_copy(z16, deg_sh.at[pl.ds(s * RPT, RPT)])
        pltpu.sync_copy(z16, odeg_sh.at[pl.ds(s * RPT, RPT)])
    plsc.subcore_barrier()

    @pl.loop(0, RT, step=4)
    def _(j0):
        descs = [
            pltpu.async_copy(table.at[gv.at[j0 + b]], rows.at[b], sems[b])
            for b in range(4)
        ]
        for b in range(4):
            descs[b].wait()
            pltpu.sync_copy(rows.at[b], acc_sh.at[sv.at[j0 + b]], add=True)
            if count:
                pltpu.sync_copy(ones_v, deg_sh.at[sv.at[j0 + b]], add=True)
                pltpu.sync_copy(ones_v, odeg_sh.at[gv.at[j0 + b]], add=True)

    plsc.subcore_barrier()
    r0 = s * RPT
    o0 = c * NPAD + r0
    pltpu.sync_copy(acc_sh.at[pl.ds(r0, RPT)], acc_out.at[pl.ds(o0, RPT)])
    if count:
        pltpu.sync_copy(deg_sh.at[pl.ds(r0, RPT)], deg_out.at[pl.ds(o0, RPT)])
        pltpu.sync_copy(odeg_sh.at[pl.ds(r0, RPT)], odeg_out.at[pl.ds(o0, RPT)])


def _make_seg_sum(d, count):
    outs = [jax.ShapeDtypeStruct((NC * NPAD, d), _f32)]
    scratch = [
        pltpu.VMEM((RT, LANES), jnp.int32),
        pltpu.VMEM((RT, LANES), jnp.int32),
        pltpu.VMEM((4, LANES, d), _f32),
        pltpu.VMEM_SHARED((NPAD, d), _f32),
    ]
    if count:
        outs += [jax.ShapeDtypeStruct((NC * NPAD, 16), _f32)] * 2
        scratch = scratch[:2] + [pltpu.VMEM((LANES, 16), _f32)] + scratch[2:]
        scratch += [pltpu.VMEM_SHARED((NPAD, 16), _f32)] * 2
    scratch += [pltpu.SemaphoreType.DMA] * 4
    return pl.kernel(
        functools.partial(_seg_sum_body, count),
        out_type=outs, mesh=_mesh(), scratch_types=scratch)


def _seg_sum_bidir_body(table, srcg, srcs, dstg, dsts, zrows, acc_out,
                        sgv, ssv, dgv, dsv, rows, acc_sh, s0, s1, s2, s3):
    c = lax.axis_index("c")
    s = lax.axis_index("s")
    w = c * NS + s
    pltpu.sync_copy(srcg.at[pl.ds(w * RT, RT)], sgv)
    pltpu.sync_copy(srcs.at[pl.ds(w * RT, RT)], ssv)
    pltpu.sync_copy(dstg.at[pl.ds(w * RT, RT)], dgv)
    pltpu.sync_copy(dsts.at[pl.ds(w * RT, RT)], dsv)
    pltpu.sync_copy(zrows, acc_sh.at[pl.ds(s * RPT, RPT)])
    plsc.subcore_barrier()

    @pl.loop(0, RT, step=2)
    def _(j0):
        d0 = pltpu.async_copy(table.at[sgv.at[j0]], rows.at[0], s0)
        d1 = pltpu.async_copy(table.at[sgv.at[j0 + 1]], rows.at[1], s1)
        d2 = pltpu.async_copy(table.at[dgv.at[j0]], rows.at[2], s2)
        d3 = pltpu.async_copy(table.at[dgv.at[j0 + 1]], rows.at[3], s3)
        d0.wait()
        pltpu.sync_copy(rows.at[0], acc_sh.at[dsv.at[j0]], add=True)
        d1.wait()
        pltpu.sync_copy(rows.at[1], acc_sh.at[dsv.at[j0 + 1]], add=True)
        d2.wait()
        pltpu.sync_copy(rows.at[2], acc_sh.at[ssv.at[j0]], add=True)
        d3.wait()
        pltpu.sync_copy(rows.at[3], acc_sh.at[ssv.at[j0 + 1]], add=True)

    plsc.subcore_barrier()
    r0 = s * RPT
    pltpu.sync_copy(acc_sh.at[pl.ds(r0, RPT)],
                    acc_out.at[pl.ds(c * NPAD + r0, RPT)])


def _make_seg_sum_bidir(d):
    scratch = [pltpu.VMEM((RT, LANES), jnp.int32) for _ in range(4)] + [
        pltpu.VMEM((4, LANES, d), _f32),
        pltpu.VMEM_SHARED((NPAD, d), _f32),
    ] + [pltpu.SemaphoreType.DMA] * 4
    return pl.kernel(
        _seg_sum_bidir_body,
        out_type=[jax.ShapeDtypeStruct((NC * NPAD, d), _f32)],
        mesh=_mesh(), scratch_types=scratch)


def _mm(a, w):
    return lax.dot_general(a, w, (((1,), (1,)), ((), ())),
                           preferred_element_type=_f32)


def _ln(x, g, b, eps=1e-5):
    mu = jnp.mean(x, axis=-1, keepdims=True)
    var = jnp.mean((x - mu) ** 2, axis=-1, keepdims=True)
    return (x - mu) / jnp.sqrt(var + eps) * g + b


def _tc1_body(x, wl1, wr1, bl1, u1_o, r1_o):
    xv = x[...]
    u1_o[...] = _mm(xv, wl1[...])
    r1_o[...] = _mm(xv, wr1[...]) + bl1[...]


def _tc2_body(acc1, degp, r1, wl2, wr2, bl2, u2_o, r2_o, dinv_o):
    a = acc1[0:N, :] + acc1[NPAD:NPAD + N, :]
    deg = degp[0:N, 0:1] + degp[NPAD:NPAD + N, 0:1]
    dinv = 1.0 / jnp.maximum(deg, 1.0)
    h = jnp.maximum(a * dinv + r1[...], 0.0)
    u2_o[...] = _mm(h, wl2[...])
    r2_o[...] = _mm(h, wr2[...]) + bl2[...]
    dinv_o[...] = dinv


def _tc3_body(acc2, dinv, r2, h2_o):
    a = acc2[0:N, :] + acc2[NPAD:NPAD + N, :]
    h2_o[...] = jnp.maximum(a * dinv[...] + r2[...], 0.0)


def _tc4_body(acc3, dinv, h2, wl3, wr3, bl3, nr_o):
    a = (acc3[0:N, :] + acc3[NPAD:NPAD + N, :]) * dinv[...]
    nr_o[...] = _mm(a, wl3[...]) + bl3[...] + _mm(h2[...], wr3[...])


def _tc5_body(nr, sp, degp, odegp, ef,
              wa1, ba1, wa2, ba2, wp, bp, gp, bpln,
              wia, wib, wic, bi, gi, biln,
              wq1a, wq1b, bq1, wq2, bq2,
              pred_o, ee_o, ie_o):
    nrv = nr[...]
    ssum = sp[0:N, :] + sp[NPAD:NPAD + N, :]
    cnt = (degp[0:N, 0:1] + degp[NPAD:NPAD + N, 0:1]
           + odegp[0:N, 0:1] + odegp[NPAD:NPAD + N, 0:1])
    eagg = nrv * ssum / jnp.maximum(cnt, 1.0)

    e = jnp.maximum(_mm(ef[...], wa1[...]) + ba1[...], 0.0)
    e = _mm(e, wa2[...]) + ba2[...]
    ee = jnp.maximum(_ln(_mm(e, wp[...]) + bp[...], gp[...], bpln[...]), 0.0)

    z = jnp.maximum(
        _mm(nrv, wia[...]) + _mm(eagg, wib[...]) + _mm(ee, wic[...]) + bi[...],
        0.0)
    ie = _ln(z, gi[...], biln[...])

    q = jnp.maximum(_mm(ee, wq1a[...]) + _mm(ie, wq1b[...]) + bq1[...], 0.0)
    logit = jnp.sum(q * wq2[...], axis=-1, keepdims=True) + bq2[...]
    pred_o[...] = jax.nn.sigmoid(logit)
    ee_o[...] = ee
    ie_o[...] = ie


def _tc_call(body, out_shapes):
    return pl.pallas_call(
        body, out_shape=[jax.ShapeDtypeStruct(s, _f32) for s in out_shapes])


def kernel(x, edge_index, explicit_features, params):
    p = params
    src = edge_index[0].astype(jnp.int32)
    dst = edge_index[1].astype(jnp.int32)
    pad_g = jnp.zeros((E_PAD - E,), jnp.int32)
    pad_s = jnp.full((E_PAD - E,), DUMMY, jnp.int32)
    src_g = jnp.concatenate([src, pad_g]).reshape(R_TOT, LANES)
    src_s = jnp.concatenate([src, pad_s]).reshape(R_TOT, LANES)
    dst_g = jnp.concatenate([dst, pad_g]).reshape(R_TOT, LANES)
    dst_s = jnp.concatenate([dst, pad_s]).reshape(R_TOT, LANES)
    zrows64 = jnp.zeros((RPT, 64), _f32)
    zrows128 = jnp.zeros((RPT, 128), _f32)
    z16 = jnp.zeros((RPT, 16), _f32)
    ones16 = jnp.ones((LANES, 16), _f32)

    row = lambda v: v.reshape(1, -1)

    u1, r1 = _tc_call(_tc1_body, [(N, 64), (N, 64)])(
        x, p['Wl1'], p['Wr1'], row(p['bl1']))
    acc1, deg_p, odeg_p = _make_seg_sum(64, True)(
        u1, src_g, dst_s, zrows64, z16, ones16)

    u2, r2, dinv = _tc_call(_tc2_body, [(N, 64), (N, 64), (N, 1)])(
        acc1, deg_p, r1, p['Wl2'], p['Wr2'], row(p['bl2']))
    (acc2,) = _make_seg_sum(64, False)(u2, src_g, dst_s, zrows64)

    (h2,) = _tc_call(_tc3_body, [(N, 64)])(acc2, dinv, r2)
    (acc3,) = _make_seg_sum(64, False)(h2, src_g, dst_s, zrows64)

    (nr,) = _tc_call(_tc4_body, [(N, 128)])(
        acc3, dinv, h2, p['Wl3'], p['Wr3'], row(p['bl3']))

    (s_p,) = _make_seg_sum_bidir(128)(nr, src_g, src_s, dst_g, dst_s, zrows128)

    wi = p['Wi']
    wq1 = p['Wq1']
    pred, ee, ie = _tc_call(_tc5_body, [(N, 1), (N, 128), (N, 128)])(
        nr, s_p, deg_p, odeg_p, explicit_features,
        p['Wa1'], row(p['ba1']), p['Wa2'], row(p['ba2']),
        p['Wp'], row(p['bp']), row(p['gp']), row(p['bp_ln']),
        wi[:, 0:128], wi[:, 128:256], wi[:, 256:384],
        row(p['bi']), row(p['gi']), row(p['bi_ln']),
        wq1[:, 0:128], wq1[:, 128:256], row(p['bq1']),
        p['Wq2'], row(p['bq2']))
    return pred, ee, ie


# speedup vs baseline: 3.7614x; 3.7614x over previous
import functools

import jax
import jax.numpy as jnp
from jax import lax
from jax.experimental import pallas as pl
from jax.experimental.pallas import tpu as pltpu
from jax.experimental.pallas import tpu_sc as plsc

N = 10000
E = 320000
NC, NS = 2, 16
NW = NC * NS
LANES = 128
RT = 80
E_PAD = NW * RT * LANES
R_TOT = E_PAD // LANES
NPAD = 10112
RPT = NPAD // NS
DUMMY = N

_f32 = jnp.float32


def _mesh():
    return plsc.VectorSubcoreMesh(
        core_axis_name="c", subcore_axis_name="s",
        num_cores=NC, num_subcores=NS)


def _seg_sum_body(count, table, gidx, sidx, zrows, *refs):
    if count:
        (z16, ones16, acc_out, deg_out, odeg_out,
         gv, sv, ones_v, rows, acc_sh, deg_sh, odeg_sh,
         s0, s1, s2, s3) = refs
    else:
        (acc_out, gv, sv, rows, acc_sh, s0, s1, s2, s3) = refs
    sems = (s0, s1, s2, s3)
    c = lax.axis_index("c")
    s = lax.axis_index("s")
    w = c * NS + s

    pltpu.sync_copy(gidx.at[pl.ds(w * RT, RT)], gv)
    pltpu.sync_copy(sidx.at[pl.ds(w * RT, RT)], sv)
    pltpu.sync_copy(zrows, acc_sh.at[pl.ds(s * RPT, RPT)])
    if count:
        pltpu.sync_copy(ones16, ones_v)
        pltpu.sync_copy(z16, deg_sh.at[pl.ds(s * RPT, RPT)])
        pltpu.sync_copy(z16, odeg_sh.at[pl.ds(s * RPT, RPT)])
    plsc.subcore_barrier()

    @pl.loop(0, RT, step=4)
    def _(j0):
        descs = [
            pltpu.async_copy(table.at[gv.at[j0 + b]], rows.at[b], sems[b])
            for b in range(4)
        ]
        for b in range(4):
            descs[b].wait()
            pltpu.sync_copy(rows.at[b], acc_sh.at[sv.at[j0 + b]], add=True)
            if count:
                pltpu.sync_copy(ones_v, deg_sh.at[sv.at[j0 + b]], add=True)
                pltpu.sync_copy(ones_v, odeg_sh.at[gv.at[j0 + b]], add=True)

    plsc.subcore_barrier()
    r0 = s * RPT
    o0 = c * NPAD + r0
    pltpu.sync_copy(acc_sh.at[pl.ds(r0, RPT)], acc_out.at[pl.ds(o0, RPT)])
    if count:
        pltpu.sync_copy(deg_sh.at[pl.ds(r0, RPT)], deg_out.at[pl.ds(o0, RPT)])
        pltpu.sync_copy(odeg_sh.at[pl.ds(r0, RPT)], odeg_out.at[pl.ds(o0, RPT)])


def _make_seg_sum(d, count):
    outs = [jax.ShapeDtypeStruct((NC * NPAD, d), _f32)]
    scratch = [
        pltpu.VMEM((RT, LANES), jnp.int32),
        pltpu.VMEM((RT, LANES), jnp.int32),
        pltpu.VMEM((4, LANES, d), _f32),
        pltpu.VMEM_SHARED((NPAD, d), _f32),
    ]
    if count:
        outs += [jax.ShapeDtypeStruct((NC * NPAD, 16), _f32)] * 2
        scratch = scratch[:2] + [pltpu.VMEM((LANES, 16), _f32)] + scratch[2:]
        scratch += [pltpu.VMEM_SHARED((NPAD, 16), _f32)] * 2
    scratch += [pltpu.SemaphoreType.DMA] * 4
    return pl.kernel(
        functools.partial(_seg_sum_body, count),
        out_type=outs, mesh=_mesh(), scratch_types=scratch,
        compiler_params=pltpu.CompilerParams(use_tc_tiling_on_sc=False))


def _seg_sum_bidir_body(table, srcg, srcs, dstg, dsts, zrows, acc_out,
                        sgv, ssv, dgv, dsv, rows, acc_sh, s0, s1, s2, s3):
    c = lax.axis_index("c")
    s = lax.axis_index("s")
    w = c * NS + s
    pltpu.sync_copy(srcg.at[pl.ds(w * RT, RT)], sgv)
    pltpu.sync_copy(srcs.at[pl.ds(w * RT, RT)], ssv)
    pltpu.sync_copy(dstg.at[pl.ds(w * RT, RT)], dgv)
    pltpu.sync_copy(dsts.at[pl.ds(w * RT, RT)], dsv)
    pltpu.sync_copy(zrows, acc_sh.at[pl.ds(s * RPT, RPT)])
    plsc.subcore_barrier()

    @pl.loop(0, RT, step=2)
    def _(j0):
        d0 = pltpu.async_copy(table.at[sgv.at[j0]], rows.at[0], s0)
        d1 = pltpu.async_copy(table.at[sgv.at[j0 + 1]], rows.at[1], s1)
        d2 = pltpu.async_copy(table.at[dgv.at[j0]], rows.at[2], s2)
        d3 = pltpu.async_copy(table.at[dgv.at[j0 + 1]], rows.at[3], s3)
        d0.wait()
        pltpu.sync_copy(rows.at[0], acc_sh.at[dsv.at[j0]], add=True)
        d1.wait()
        pltpu.sync_copy(rows.at[1], acc_sh.at[dsv.at[j0 + 1]], add=True)
        d2.wait()
        pltpu.sync_copy(rows.at[2], acc_sh.at[ssv.at[j0]], add=True)
        d3.wait()
        pltpu.sync_copy(rows.at[3], acc_sh.at[ssv.at[j0 + 1]], add=True)

    plsc.subcore_barrier()
    r0 = s * RPT
    pltpu.sync_copy(acc_sh.at[pl.ds(r0, RPT)],
                    acc_out.at[pl.ds(c * NPAD + r0, RPT)])


def _make_seg_sum_bidir(d):
    scratch = [pltpu.VMEM((RT, LANES), jnp.int32) for _ in range(4)] + [
        pltpu.VMEM((4, LANES, d), _f32),
        pltpu.VMEM_SHARED((NPAD, d), _f32),
    ] + [pltpu.SemaphoreType.DMA] * 4
    return pl.kernel(
        _seg_sum_bidir_body,
        out_type=[jax.ShapeDtypeStruct((NC * NPAD, d), _f32)],
        mesh=_mesh(), scratch_types=scratch,
        compiler_params=pltpu.CompilerParams(use_tc_tiling_on_sc=False))


def _mm(a, w):
    return lax.dot_general(a, w, (((1,), (1,)), ((), ())),
                           preferred_element_type=_f32)


def _ln(x, g, b, eps=1e-5):
    mu = jnp.mean(x, axis=-1, keepdims=True)
    var = jnp.mean((x - mu) ** 2, axis=-1, keepdims=True)
    return (x - mu) / jnp.sqrt(var + eps) * g + b


def _tc1_body(x, wl1, wr1, bl1, u1_o, r1_o):
    xv = x[...]
    u1_o[...] = _mm(xv, wl1[...])
    r1_o[...] = _mm(xv, wr1[...]) + bl1[...]


def _tc2_body(acc1, degp, r1, wl2, wr2, bl2, u2_o, r2_o, dinv_o):
    a = acc1[0:N, :] + acc1[NPAD:NPAD + N, :]
    deg = degp[0:N, 0:1] + degp[NPAD:NPAD + N, 0:1]
    dinv = 1.0 / jnp.maximum(deg, 1.0)
    h = jnp.maximum(a * dinv + r1[...], 0.0)
    u2_o[...] = _mm(h, wl2[...])
    r2_o[...] = _mm(h, wr2[...]) + bl2[...]
    dinv_o[...] = dinv


def _tc3_body(acc2, dinv, r2, h2_o):
    a = acc2[0:N, :] + acc2[NPAD:NPAD + N, :]
    h2_o[...] = jnp.maximum(a * dinv[...] + r2[...], 0.0)


def _tc4_body(acc3, dinv, h2, wl3, wr3, bl3, nr_o, nrlo_o, nrhi_o):
    a = (acc3[0:N, :] + acc3[NPAD:NPAD + N, :]) * dinv[...]
    nr = _mm(a, wl3[...]) + bl3[...] + _mm(h2[...], wr3[...])
    nr_o[...] = nr
    nrlo_o[...] = nr[:, 0:64]
    nrhi_o[...] = nr[:, 64:128]


def _tcs_body(sp_lo, sp_hi, degp, odegp, slo_o, shi_o, cinv_o):
    slo_o[...] = sp_lo[0:N, :] + sp_lo[NPAD:NPAD + N, :]
    shi_o[...] = sp_hi[0:N, :] + sp_hi[NPAD:NPAD + N, :]
    cnt = (degp[0:N, 0:1] + degp[NPAD:NPAD + N, 0:1]
           + odegp[0:N, 0:1] + odegp[NPAD:NPAD + N, 0:1])
    cinv_o[...] = 1.0 / jnp.maximum(cnt, 1.0)


def _tc5_body(nr, s_lo_r, s_hi_r, cinv_r, ef,
              wa1, ba1, wa2, ba2, wp, bp, gp, bpln,
              wia, wib_lo, wib_hi, wic, bi, gi, biln,
              wq1a, wq1b, bq1, wq2, bq2,
              pred_o, ee_o, ie_o):
    nrv = nr[...]
    cinv = cinv_r[...]
    eagg_lo = nrv[:, 0:64] * s_lo_r[...] * cinv
    eagg_hi = nrv[:, 64:128] * s_hi_r[...] * cinv

    e = jnp.maximum(_mm(ef[...], wa1[...]) + ba1[...], 0.0)
    e = _mm(e, wa2[...]) + ba2[...]
    ee = jnp.maximum(_ln(_mm(e, wp[...]) + bp[...], gp[...], bpln[...]), 0.0)

    z = jnp.maximum(
        _mm(nrv, wia[...]) + _mm(eagg_lo, wib_lo[...])
        + _mm(eagg_hi, wib_hi[...]) + _mm(ee, wic[...]) + bi[...],
        0.0)
    ie = _ln(z, gi[...], biln[...])

    q = jnp.maximum(_mm(ee, wq1a[...]) + _mm(ie, wq1b[...]) + bq1[...], 0.0)
    logit = jnp.sum(q * wq2[...], axis=-1, keepdims=True) + bq2[...]
    pred_o[...] = jax.nn.sigmoid(logit)
    ee_o[...] = ee
    ie_o[...] = ie


def _tc_call(body, out_shapes):
    return pl.pallas_call(
        body, out_shape=[jax.ShapeDtypeStruct(s, _f32) for s in out_shapes])


B5 = 2000


def _tc5_call(n_in, out_shapes):
    def spec(shape):
        if shape[0] == N:
            return pl.BlockSpec((B5, shape[1]), lambda i: (i, 0))
        return pl.BlockSpec(shape, lambda i: (0, 0))

    def wrap(*arrays):
        in_specs = [spec(a.shape) for a in arrays]
        return pl.pallas_call(
            _tc5_body,
            grid=(N // B5,),
            in_specs=in_specs,
            out_specs=[pl.BlockSpec((B5, s[1]), lambda i: (i, 0))
                       for s in out_shapes],
            out_shape=[jax.ShapeDtypeStruct(s, _f32) for s in out_shapes],
        )(*arrays)
    return wrap


def kernel(x, edge_index, explicit_features, params):
    p = params
    src = edge_index[0].astype(jnp.int32)
    dst = edge_index[1].astype(jnp.int32)
    pad_g = jnp.zeros((E_PAD - E,), jnp.int32)
    pad_s = jnp.full((E_PAD - E,), DUMMY, jnp.int32)
    src_g = jnp.concatenate([src, pad_g]).reshape(R_TOT, LANES)
    src_s = jnp.concatenate([src, pad_s]).reshape(R_TOT, LANES)
    dst_g = jnp.concatenate([dst, pad_g]).reshape(R_TOT, LANES)
    dst_s = jnp.concatenate([dst, pad_s]).reshape(R_TOT, LANES)
    zrows64 = jnp.zeros((RPT, 64), _f32)
    zrows128 = jnp.zeros((RPT, 128), _f32)
    z16 = jnp.zeros((RPT, 16), _f32)
    ones16 = jnp.ones((LANES, 16), _f32)

    row = lambda v: v.reshape(1, -1)

    u1, r1 = _tc_call(_tc1_body, [(N, 64), (N, 64)])(
        x, p['Wl1'], p['Wr1'], row(p['bl1']))
    acc1, deg_p, odeg_p = _make_seg_sum(64, True)(
        u1, src_g, dst_s, zrows64, z16, ones16)

    u2, r2, dinv = _tc_call(_tc2_body, [(N, 64), (N, 64), (N, 1)])(
        acc1, deg_p, r1, p['Wl2'], p['Wr2'], row(p['bl2']))
    (acc2,) = _make_seg_sum(64, False)(u2, src_g, dst_s, zrows64)

    (h2,) = _tc_call(_tc3_body, [(N, 64)])(acc2, dinv, r2)
    (acc3,) = _make_seg_sum(64, False)(h2, src_g, dst_s, zrows64)

    nr, nr_lo, nr_hi = _tc_call(_tc4_body, [(N, 128), (N, 64), (N, 64)])(
        acc3, dinv, h2, p['Wl3'], p['Wr3'], row(p['bl3']))

    bidir = _make_seg_sum_bidir(64)
    (s_p_lo,) = bidir(nr_lo, src_g, src_s, dst_g, dst_s, zrows64)
    (s_p_hi,) = bidir(nr_hi, src_g, src_s, dst_g, dst_s, zrows64)

    s_lo, s_hi, cinv = _tc_call(_tcs_body, [(N, 64), (N, 64), (N, 1)])(
        s_p_lo, s_p_hi, deg_p, odeg_p)

    wi = p['Wi']
    wq1 = p['Wq1']
    pred, ee, ie = _tc5_call(None, [(N, 1), (N, 128), (N, 128)])(
        nr, s_lo, s_hi, cinv, explicit_features,
        p['Wa1'], row(p['ba1']), p['Wa2'], row(p['ba2']),
        p['Wp'], row(p['bp']), row(p['gp']), row(p['bp_ln']),
        wi[:, 0:128], wi[:, 128:192], wi[:, 192:256], wi[:, 256:384],
        row(p['bi']), row(p['gi']), row(p['bi_ln']),
        wq1[:, 0:128], wq1[:, 128:256], row(p['bq1']),
        p['Wq2'], row(p['bq2']))
    return pred, ee, ie


# speedup vs baseline: 3.9742x; 1.0566x over previous
import functools

import jax
import jax.numpy as jnp
from jax import lax
from jax.experimental import pallas as pl
from jax.experimental.pallas import tpu as pltpu
from jax.experimental.pallas import tpu_sc as plsc

N = 10000
E = 320000
NC, NS = 2, 16
NW = NC * NS
LANES = 128
RT = 80
E_PAD = NW * RT * LANES
R_TOT = E_PAD // LANES
NPAD = 10112
RPT = NPAD // NS
DUMMY = N

_f32 = jnp.float32


def _mesh():
    return plsc.VectorSubcoreMesh(
        core_axis_name="c", subcore_axis_name="s",
        num_cores=NC, num_subcores=NS)


def _seg_sum_body(count, table, gidx, sidx, zrows, *refs):
    if count:
        (z16, ones16, acc_out, deg_out, odeg_out,
         gv, sv, ones_v, rows, acc_sh, deg_sh, odeg_sh,
         s0, s1, s2, s3) = refs
    else:
        (acc_out, gv, sv, rows, acc_sh, s0, s1, s2, s3) = refs
    sems = (s0, s1, s2, s3)
    c = lax.axis_index("c")
    s = lax.axis_index("s")
    w = c * NS + s

    pltpu.sync_copy(gidx.at[pl.ds(w * RT, RT)], gv)
    pltpu.sync_copy(sidx.at[pl.ds(w * RT, RT)], sv)
    pltpu.sync_copy(zrows, acc_sh.at[pl.ds(s * RPT, RPT)])
    if count:
        pltpu.sync_copy(ones16, ones_v)
        pltpu.sync_copy(z16, deg_sh.at[pl.ds(s * RPT, RPT)])
        pltpu.sync_copy(z16, odeg_sh.at[pl.ds(s * RPT, RPT)])
    plsc.subcore_barrier()

    @pl.loop(0, RT, step=4)
    def _(j0):
        descs = [
            pltpu.async_copy(table.at[gv.at[j0 + b]], rows.at[b], sems[b])
            for b in range(4)
        ]
        for b in range(4):
            descs[b].wait()
            pltpu.sync_copy(rows.at[b], acc_sh.at[sv.at[j0 + b]], add=True)
            if count:
                pltpu.sync_copy(ones_v, deg_sh.at[sv.at[j0 + b]], add=True)
                pltpu.sync_copy(ones_v, odeg_sh.at[gv.at[j0 + b]], add=True)

    plsc.subcore_barrier()
    r0 = s * RPT
    o0 = c * NPAD + r0
    pltpu.sync_copy(acc_sh.at[pl.ds(r0, RPT)], acc_out.at[pl.ds(o0, RPT)])
    if count:
        pltpu.sync_copy(deg_sh.at[pl.ds(r0, RPT)], deg_out.at[pl.ds(o0, RPT)])
        pltpu.sync_copy(odeg_sh.at[pl.ds(r0, RPT)], odeg_out.at[pl.ds(o0, RPT)])


def _wait_like(table, rows_slot, sem):
    pltpu.make_async_copy(table.at[pl.ds(0, LANES)], rows_slot, sem).wait()


def _seg_sum_async_body(table, gidx, sidx, zrows, acc_out,
                        gv, sv, rows, acc_sh, *sems):
    semg, sems_ = sems[:8], sems[8:16]
    c = lax.axis_index("c")
    s = lax.axis_index("s")
    w = c * NS + s
    pltpu.sync_copy(gidx.at[pl.ds(w * RT, RT)], gv)
    pltpu.sync_copy(sidx.at[pl.ds(w * RT, RT)], sv)
    pltpu.sync_copy(zrows, acc_sh.at[pl.ds(s * RPT, RPT)])
    plsc.subcore_barrier()

    for b in range(4):
        pltpu.async_copy(table.at[gv.at[b]], rows.at[b], semg[b])

    @pl.loop(0, RT, step=8)
    def _(j0):
        for b in range(8):
            j = j0 + b
            sl4 = (b + 4) % 8
            _wait_like(table, rows.at[b], semg[b])
            pltpu.async_copy(rows.at[b], acc_sh.at[sv.at[j]], sems_[b],
                             add=True)

            @pl.when(j >= 4)
            def _():
                _wait_like(table, rows.at[sl4], sems_[sl4])

            @pl.when(j + 4 < RT)
            def _():
                pltpu.async_copy(table.at[gv.at[j + 4]], rows.at[sl4],
                                 semg[sl4])

    for sl in (4, 5, 6, 7):
        _wait_like(table, rows.at[sl], sems_[sl])
    plsc.subcore_barrier()
    r0 = s * RPT
    pltpu.sync_copy(acc_sh.at[pl.ds(r0, RPT)],
                    acc_out.at[pl.ds(c * NPAD + r0, RPT)])


def _make_seg_sum_async(d):
    scratch = [
        pltpu.VMEM((RT, LANES), jnp.int32),
        pltpu.VMEM((RT, LANES), jnp.int32),
        pltpu.VMEM((8, LANES, d), _f32),
        pltpu.VMEM_SHARED((NPAD, d), _f32),
    ] + [pltpu.SemaphoreType.DMA] * 16
    return pl.kernel(
        _seg_sum_async_body,
        out_type=[jax.ShapeDtypeStruct((NC * NPAD, d), _f32)],
        mesh=_mesh(), scratch_types=scratch,
        compiler_params=pltpu.CompilerParams(use_tc_tiling_on_sc=False))


def _make_seg_sum(d, count):
    outs = [jax.ShapeDtypeStruct((NC * NPAD, d), _f32)]
    scratch = [
        pltpu.VMEM((RT, LANES), jnp.int32),
        pltpu.VMEM((RT, LANES), jnp.int32),
        pltpu.VMEM((4, LANES, d), _f32),
        pltpu.VMEM_SHARED((NPAD, d), _f32),
    ]
    if count:
        outs += [jax.ShapeDtypeStruct((NC * NPAD, 16), _f32)] * 2
        scratch = scratch[:2] + [pltpu.VMEM((LANES, 16), _f32)] + scratch[2:]
        scratch += [pltpu.VMEM_SHARED((NPAD, 16), _f32)] * 2
    scratch += [pltpu.SemaphoreType.DMA] * 4
    return pl.kernel(
        functools.partial(_seg_sum_body, count),
        out_type=outs, mesh=_mesh(), scratch_types=scratch,
        compiler_params=pltpu.CompilerParams(use_tc_tiling_on_sc=False))


def _seg_sum_bidir_body(table, srcg, srcs, dstg, dsts, zrows, acc_out,
                        sgv, ssv, dgv, dsv, rows, acc_sh, *sems):
    semg, sems_ = sems[:6], sems[6:12]
    c = lax.axis_index("c")
    s = lax.axis_index("s")
    w = c * NS + s
    pltpu.sync_copy(srcg.at[pl.ds(w * RT, RT)], sgv)
    pltpu.sync_copy(srcs.at[pl.ds(w * RT, RT)], ssv)
    pltpu.sync_copy(dstg.at[pl.ds(w * RT, RT)], dgv)
    pltpu.sync_copy(dsts.at[pl.ds(w * RT, RT)], dsv)
    pltpu.sync_copy(zrows, acc_sh.at[pl.ds(s * RPT, RPT)])
    plsc.subcore_barrier()

    gref = (sgv, dgv)
    sref = (dsv, ssv)

    pltpu.async_copy(table.at[sgv.at[0]], rows.at[0], semg[0])
    pltpu.async_copy(table.at[dgv.at[0]], rows.at[1], semg[1])
    pltpu.async_copy(table.at[sgv.at[1]], rows.at[2], semg[2])

    def emit(j, b, d):
        static = isinstance(j, int)
        sl = (2 * b + d) % 6
        sl3 = (sl + 3) % 6
        _wait_like(table, rows.at[sl], semg[sl])
        pltpu.async_copy(rows.at[sl], acc_sh.at[sref[d].at[j]], sems_[sl],
                         add=True)

        def wait_prev():
            _wait_like(table, rows.at[sl3], sems_[sl3])

        def prefetch():
            pltpu.async_copy(table.at[gref[1 - d].at[j + 1 + d]],
                             rows.at[sl3], semg[sl3])

        if static:
            if 2 * j + d >= 3:
                wait_prev()
            if j + 1 + d < RT:
                prefetch()
        else:
            pl.when(2 * j + d >= 3)(wait_prev)
            pl.when(j + 1 + d < RT)(prefetch)

    @pl.loop(0, RT - 2, step=3)
    def _(j0):
        for b in range(3):
            for d in range(2):
                emit(j0 + b, b, d)

    for b in range(2):
        for d in range(2):
            emit(RT - 2 + b, b, d)

    for sl in (1, 2, 3):
        _wait_like(table, rows.at[sl], sems_[sl])
    plsc.subcore_barrier()
    r0 = s * RPT
    pltpu.sync_copy(acc_sh.at[pl.ds(r0, RPT)],
                    acc_out.at[pl.ds(c * NPAD + r0, RPT)])


def _make_seg_sum_bidir(d):
    scratch = [pltpu.VMEM((RT, LANES), jnp.int32) for _ in range(4)] + [
        pltpu.VMEM((6, LANES, d), _f32),
        pltpu.VMEM_SHARED((NPAD, d), _f32),
    ] + [pltpu.SemaphoreType.DMA] * 12
    return pl.kernel(
        _seg_sum_bidir_body,
        out_type=[jax.ShapeDtypeStruct((NC * NPAD, d), _f32)],
        mesh=_mesh(), scratch_types=scratch,
        compiler_params=pltpu.CompilerParams(use_tc_tiling_on_sc=False))


def _mm(a, w):
    return lax.dot_general(a, w, (((1,), (1,)), ((), ())),
                           preferred_element_type=_f32)


def _ln(x, g, b, eps=1e-5):
    mu = jnp.mean(x, axis=-1, keepdims=True)
    var = jnp.mean((x - mu) ** 2, axis=-1, keepdims=True)
    return (x - mu) / jnp.sqrt(var + eps) * g + b


def _tc1_body(x, wl1, wr1, bl1, u1_o, r1_o):
    xv = x[...]
    u1_o[...] = _mm(xv, wl1[...])
    r1_o[...] = _mm(xv, wr1[...]) + bl1[...]


def _tc2_body(acc1, degp, r1, wl2, wr2, bl2, u2_o, r2_o, dinv_o):
    a = acc1[0:N, :] + acc1[NPAD:NPAD + N, :]
    deg = degp[0:N, 0:1] + degp[NPAD:NPAD + N, 0:1]
    dinv = 1.0 / jnp.maximum(deg, 1.0)
    h = jnp.maximum(a * dinv + r1[...], 0.0)
    u2_o[...] = _mm(h, wl2[...])
    r2_o[...] = _mm(h, wr2[...]) + bl2[...]
    dinv_o[...] = dinv


def _tc3_body(acc2, dinv, r2, h2_o):
    a = acc2[0:N, :] + acc2[NPAD:NPAD + N, :]
    h2_o[...] = jnp.maximum(a * dinv[...] + r2[...], 0.0)


def _tc4_body(acc3, dinv, h2, wl3, wr3, bl3, nr_o, nrlo_o, nrhi_o):
    a = (acc3[0:N, :] + acc3[NPAD:NPAD + N, :]) * dinv[...]
    nr = _mm(a, wl3[...]) + bl3[...] + _mm(h2[...], wr3[...])
    nr_o[...] = nr
    nrlo_o[...] = nr[:, 0:64]
    nrhi_o[...] = nr[:, 64:128]


def _tcs_body(sp_lo, sp_hi, degp, odegp, slo_o, shi_o, cinv_o):
    slo_o[...] = sp_lo[0:N, :] + sp_lo[NPAD:NPAD + N, :]
    shi_o[...] = sp_hi[0:N, :] + sp_hi[NPAD:NPAD + N, :]
    cnt = (degp[0:N, 0:1] + degp[NPAD:NPAD + N, 0:1]
           + odegp[0:N, 0:1] + odegp[NPAD:NPAD + N, 0:1])
    cinv_o[...] = 1.0 / jnp.maximum(cnt, 1.0)


def _tc5_body(nr, s_lo_r, s_hi_r, cinv_r, ef,
              wa1, ba1, wa2, ba2, wp, bp, gp, bpln,
              wia, wib_lo, wib_hi, wic, bi, gi, biln,
              wq1a, wq1b, bq1, wq2, bq2,
              pred_o, ee_o, ie_o):
    nrv = nr[...]
    cinv = cinv_r[...]
    eagg_lo = nrv[:, 0:64] * s_lo_r[...] * cinv
    eagg_hi = nrv[:, 64:128] * s_hi_r[...] * cinv

    e = jnp.maximum(_mm(ef[...], wa1[...]) + ba1[...], 0.0)
    e = _mm(e, wa2[...]) + ba2[...]
    ee = jnp.maximum(_ln(_mm(e, wp[...]) + bp[...], gp[...], bpln[...]), 0.0)

    z = jnp.maximum(
        _mm(nrv, wia[...]) + _mm(eagg_lo, wib_lo[...])
        + _mm(eagg_hi, wib_hi[...]) + _mm(ee, wic[...]) + bi[...],
        0.0)
    ie = _ln(z, gi[...], biln[...])

    q = jnp.maximum(_mm(ee, wq1a[...]) + _mm(ie, wq1b[...]) + bq1[...], 0.0)
    logit = jnp.sum(q * wq2[...], axis=-1, keepdims=True) + bq2[...]
    pred_o[...] = jax.nn.sigmoid(logit)
    ee_o[...] = ee
    ie_o[...] = ie


def _tc_call(body, out_shapes):
    return pl.pallas_call(
        body, out_shape=[jax.ShapeDtypeStruct(s, _f32) for s in out_shapes])


B5 = 2000


def _tc5_call(n_in, out_shapes):
    def spec(shape):
        if shape[0] == N:
            return pl.BlockSpec((B5, shape[1]), lambda i: (i, 0))
        return pl.BlockSpec(shape, lambda i: (0, 0))

    def wrap(*arrays):
        in_specs = [spec(a.shape) for a in arrays]
        return pl.pallas_call(
            _tc5_body,
            grid=(N // B5,),
            in_specs=in_specs,
            out_specs=[pl.BlockSpec((B5, s[1]), lambda i: (i, 0))
                       for s in out_shapes],
            out_shape=[jax.ShapeDtypeStruct(s, _f32) for s in out_shapes],
        )(*arrays)
    return wrap


def kernel(x, edge_index, explicit_features, params):
    p = params
    src = edge_index[0].astype(jnp.int32)
    dst = edge_index[1].astype(jnp.int32)
    pad_g = jnp.zeros((E_PAD - E,), jnp.int32)
    pad_s = jnp.full((E_PAD - E,), DUMMY, jnp.int32)
    src_g = jnp.concatenate([src, pad_g]).reshape(R_TOT, LANES)
    src_s = jnp.concatenate([src, pad_s]).reshape(R_TOT, LANES)
    dst_g = jnp.concatenate([dst, pad_g]).reshape(R_TOT, LANES)
    dst_s = jnp.concatenate([dst, pad_s]).reshape(R_TOT, LANES)
    zrows64 = jnp.zeros((RPT, 64), _f32)
    zrows128 = jnp.zeros((RPT, 128), _f32)
    z16 = jnp.zeros((RPT, 16), _f32)
    ones16 = jnp.ones((LANES, 16), _f32)

    row = lambda v: v.reshape(1, -1)

    u1, r1 = _tc_call(_tc1_body, [(N, 64), (N, 64)])(
        x, p['Wl1'], p['Wr1'], row(p['bl1']))
    acc1, deg_p, odeg_p = _make_seg_sum(64, True)(
        u1, src_g, dst_s, zrows64, z16, ones16)

    u2, r2, dinv = _tc_call(_tc2_body, [(N, 64), (N, 64), (N, 1)])(
        acc1, deg_p, r1, p['Wl2'], p['Wr2'], row(p['bl2']))
    seg64 = _make_seg_sum_async(64)
    (acc2,) = seg64(u2, src_g, dst_s, zrows64)

    (h2,) = _tc_call(_tc3_body, [(N, 64)])(acc2, dinv, r2)
    (acc3,) = seg64(h2, src_g, dst_s, zrows64)

    nr, nr_lo, nr_hi = _tc_call(_tc4_body, [(N, 128), (N, 64), (N, 64)])(
        acc3, dinv, h2, p['Wl3'], p['Wr3'], row(p['bl3']))

    bidir = _make_seg_sum_bidir(64)
    (s_p_lo,) = bidir(nr_lo, src_g, src_s, dst_g, dst_s, zrows64)
    (s_p_hi,) = bidir(nr_hi, src_g, src_s, dst_g, dst_s, zrows64)

    s_lo, s_hi, cinv = _tc_call(_tcs_body, [(N, 64), (N, 64), (N, 1)])(
        s_p_lo, s_p_hi, deg_p, odeg_p)

    wi = p['Wi']
    wq1 = p['Wq1']
    pred, ee, ie = _tc5_call(None, [(N, 1), (N, 128), (N, 128)])(
        nr, s_lo, s_hi, cinv, explicit_features,
        p['Wa1'], row(p['ba1']), p['Wa2'], row(p['ba2']),
        p['Wp'], row(p['bp']), row(p['gp']), row(p['bp_ln']),
        wi[:, 0:128], wi[:, 128:192], wi[:, 192:256], wi[:, 256:384],
        row(p['bi']), row(p['gi']), row(p['bi_ln']),
        wq1[:, 0:128], wq1[:, 128:256], row(p['bq1']),
        p['Wq2'], row(p['bq2']))
    return pred, ee, ie


# speedup vs baseline: 4.1427x; 1.0424x over previous
import jax
import jax.numpy as jnp
from jax import lax
from jax.experimental import pallas as pl
from jax.experimental.pallas import tpu as pltpu
from jax.experimental.pallas import tpu_sc as plsc

N = 10000
N16 = 10016
E = 320000
NC, NS = 2, 16
LANES = 128
RT_BIG = 128
RT_SMALL = 32
BIG_CORE = 0
R_TOT = NS * (RT_BIG + RT_SMALL)
OFF_SMALL = NS * RT_BIG
R_ALLOC = R_TOT
E_PAD = R_ALLOC * LANES
NPAD = 10112
RPT = NPAD // NS
DUMMY = N

_f32 = jnp.float32


def _mesh():
    return plsc.VectorSubcoreMesh(
        core_axis_name="c", subcore_axis_name="s",
        num_cores=NC, num_subcores=NS)


def _sc_params():
    return pltpu.CompilerParams(use_tc_tiling_on_sc=False)


def _core_share(c, s):
    big = c == BIG_CORE
    rt = jnp.where(big, RT_BIG, RT_SMALL)
    base = jnp.where(big, s * RT_BIG, OFF_SMALL + s * RT_SMALL)
    return big, rt, base


def _stage_idx(big, base, hbm, vbuf):
    @pl.when(big)
    def _():
        pltpu.sync_copy(hbm.at[pl.ds(base, RT_BIG)], vbuf)

    @pl.when(jnp.logical_not(big))
    def _():
        pltpu.sync_copy(hbm.at[pl.ds(base, RT_SMALL)],
                        vbuf.at[pl.ds(0, RT_SMALL)])


def _wait_like(table, rows_slot, sem):
    pltpu.make_async_copy(table.at[pl.ds(0, LANES)], rows_slot, sem).wait()


def _seg_sum_body(count, table, gidx, sidx, zrows, *refs):
    if count:
        (z16, ones16, acc_out, deg_out, odeg_out,
         gv, sv, ones_v, rows, acc_sh, deg_sh, odeg_sh, *sems) = refs
        nslot, dist = 4, 2
        semg, sems_ = sems[:4], sems[4:8]
        sem_oi, sem_oo = sems[8], sems[9]
    else:
        (acc_out, gv, sv, rows, acc_sh, *sems) = refs
        nslot, dist = 6, 3
        semg, sems_ = sems[:6], sems[6:12]
    c = lax.axis_index("c")
    s = lax.axis_index("s")
    big, rt, base = _core_share(c, s)

    _stage_idx(big, base, gidx, gv)
    _stage_idx(big, base, sidx, sv)
    pltpu.sync_copy(zrows, acc_sh.at[pl.ds(s * RPT, RPT)])
    if count:
        pltpu.sync_copy(ones16, ones_v)
        pltpu.sync_copy(z16, deg_sh.at[pl.ds(s * RPT, RPT)])
        pltpu.sync_copy(z16, odeg_sh.at[pl.ds(s * RPT, RPT)])
    plsc.subcore_barrier()

    for b in range(dist):
        pltpu.async_copy(table.at[gv.at[b]], rows.at[b], semg[b])

    def emit(j, sl):
        sl_n = (sl + dist) % nslot
        _wait_like(table, rows.at[sl], semg[sl])
        pltpu.async_copy(rows.at[sl], acc_sh.at[sv.at[j]], sems_[sl],
                         add=True)
        if count:
            pltpu.async_copy(ones_v, deg_sh.at[sv.at[j]], sem_oi, add=True)
            pltpu.async_copy(ones_v, odeg_sh.at[gv.at[j]], sem_oo, add=True)

        @pl.when(j >= dist)
        def _():
            _wait_like(table, rows.at[sl_n], sems_[sl_n])
            if count:
                _wait_like(ones16, ones_v, sem_oi)
                _wait_like(ones16, ones_v, sem_oo)

        @pl.when(j + dist < rt)
        def _():
            pltpu.async_copy(table.at[gv.at[j + dist]], rows.at[sl_n],
                             semg[sl_n])

    if count:
        @pl.loop(0, rt, step=4)
        def _(j0):
            for b in range(4):
                emit(j0 + b, b)
    else:
        @pl.loop(0, rt - 2, step=6)
        def _(j0):
            for b in range(6):
                emit(j0 + b, b)
        emit(rt - 2, 0)
        emit(rt - 1, 1)

    drain = (2, 3) if count else (5, 0, 1)
    for sl in drain:
        _wait_like(table, rows.at[sl], sems_[sl])
    if count:
        for sem in (sem_oi, sem_oo):
            _wait_like(ones16, ones_v, sem)
            _wait_like(ones16, ones_v, sem)
    plsc.subcore_barrier()
    r0 = s * RPT
    o0 = c * NPAD + r0
    pltpu.sync_copy(acc_sh.at[pl.ds(r0, RPT)], acc_out.at[pl.ds(o0, RPT)])
    if count:
        pltpu.sync_copy(deg_sh.at[pl.ds(r0, RPT)], deg_out.at[pl.ds(o0, RPT)])
        pltpu.sync_copy(odeg_sh.at[pl.ds(r0, RPT)], odeg_out.at[pl.ds(o0, RPT)])


def _make_seg_sum(d, count):
    nslot = 4 if count else 6
    outs = [jax.ShapeDtypeStruct((NC * NPAD, d), _f32)]
    scratch = [
        pltpu.VMEM((RT_BIG, LANES), jnp.int32),
        pltpu.VMEM((RT_BIG, LANES), jnp.int32),
        pltpu.VMEM((nslot, LANES, d), _f32),
        pltpu.VMEM_SHARED((NPAD, d), _f32),
    ]
    nsem = 2 * nslot
    if count:
        outs += [jax.ShapeDtypeStruct((NC * NPAD, 16), _f32)] * 2
        scratch = scratch[:2] + [pltpu.VMEM((LANES, 16), _f32)] + scratch[2:]
        scratch += [pltpu.VMEM_SHARED((NPAD, 16), _f32)] * 2
        nsem += 2
    scratch += [pltpu.SemaphoreType.DMA] * nsem

    def body(*refs):
        _seg_sum_body(count, *refs)

    return pl.kernel(
        body, out_type=outs, mesh=_mesh(), scratch_types=scratch,
        compiler_params=_sc_params())


def _seg_sum_bidir_body(table, srci, dsti, zrows, acc_out,
                        sv_, dv_, rows, acc_sh, *sems):
    semg, sems_ = sems[:6], sems[6:12]
    c = lax.axis_index("c")
    s = lax.axis_index("s")
    big, rt, base = _core_share(c, s)
    _stage_idx(big, base, srci, sv_)
    _stage_idx(big, base, dsti, dv_)
    pltpu.sync_copy(zrows, acc_sh.at[pl.ds(s * RPT, RPT)])
    plsc.subcore_barrier()

    gref = (sv_, dv_)
    sref = (dv_, sv_)

    pltpu.async_copy(table.at[sv_.at[0]], rows.at[0], semg[0])
    pltpu.async_copy(table.at[dv_.at[0]], rows.at[1], semg[1])
    pltpu.async_copy(table.at[sv_.at[1]], rows.at[2], semg[2])

    def emit(j, sl, d):
        sl3 = (sl + 3) % 6
        _wait_like(table, rows.at[sl], semg[sl])
        pltpu.async_copy(rows.at[sl], acc_sh.at[sref[d].at[j]], sems_[sl],
                         add=True)

        @pl.when(2 * j + d >= 3)
        def _():
            _wait_like(table, rows.at[sl3], sems_[sl3])

        @pl.when(j + 1 + d < rt)
        def _():
            pltpu.async_copy(table.at[gref[1 - d].at[j + 1 + d]],
                             rows.at[sl3], semg[sl3])

    @pl.loop(0, rt - 2, step=3)
    def _(j0):
        for b in range(3):
            for d in range(2):
                emit(j0 + b, (2 * b + d) % 6, d)

    for b in range(2):
        for d in range(2):
            emit(rt - 2 + b, 2 * b + d, d)

    for sl in (1, 2, 3):
        _wait_like(table, rows.at[sl], sems_[sl])
    plsc.subcore_barrier()
    r0 = s * RPT
    pltpu.sync_copy(acc_sh.at[pl.ds(r0, RPT)],
                    acc_out.at[pl.ds(c * NPAD + r0, RPT)])


def _make_seg_sum_bidir(d):
    scratch = [pltpu.VMEM((RT_BIG, LANES), jnp.int32) for _ in range(2)] + [
        pltpu.VMEM((6, LANES, d), _f32),
        pltpu.VMEM_SHARED((NPAD, d), _f32),
    ] + [pltpu.SemaphoreType.DMA] * 12
    return pl.kernel(
        _seg_sum_bidir_body,
        out_type=[jax.ShapeDtypeStruct((NC * NPAD, d), _f32)],
        mesh=_mesh(), scratch_types=scratch,
        compiler_params=_sc_params())


def _mm(a, w):
    return lax.dot_general(a, w, (((1,), (1,)), ((), ())),
                           preferred_element_type=_f32)


def _ln(x, g, b, eps=1e-5):
    mu = jnp.mean(x, axis=-1, keepdims=True)
    var = jnp.mean((x - mu) ** 2, axis=-1, keepdims=True)
    return (x - mu) / jnp.sqrt(var + eps) * g + b


def _pad16(x):
    return jnp.concatenate([x, jnp.zeros((N16 - N, x.shape[1]), _f32)], 0)


def _tc1_body(x, wl1, wr1, bl1, u1_o, r1_o):
    xv = x[...]
    u1_o[...] = _pad16(_mm(xv, wl1[...]))
    r1_o[...] = _mm(xv, wr1[...]) + bl1[...]


def _tc2_body(acc1, degp, r1, wl2, wr2, bl2, u2_o, r2_o, dinv_o):
    a = acc1[0:N, :] + acc1[NPAD:NPAD + N, :]
    deg = degp[0:N, 0:1] + degp[NPAD:NPAD + N, 0:1]
    dinv = 1.0 / jnp.maximum(deg, 1.0)
    h = jnp.maximum(a * dinv + r1[...], 0.0)
    u2_o[...] = _pad16(_mm(h, wl2[...]))
    r2_o[...] = _mm(h, wr2[...]) + bl2[...]
    dinv_o[...] = dinv


def _tc3_body(acc2, dinv, r2, h2_o):
    a = acc2[0:N, :] + acc2[NPAD:NPAD + N, :]
    h2_o[...] = _pad16(jnp.maximum(a * dinv[...] + r2[...], 0.0))


def _tc4_body(acc3, dinv, h2, wl3, wr3, bl3, nr_o, nrlo_o, nrhi_o):
    h2v = h2[0:N, :]
    a = (acc3[0:N, :] + acc3[NPAD:NPAD + N, :]) * dinv[...]
    nr = _mm(a, wl3[...]) + bl3[...] + _mm(h2v, wr3[...])
    nr_o[...] = nr
    nrlo_o[...] = _pad16(nr[:, 0:64])
    nrhi_o[...] = _pad16(nr[:, 64:128])


def _tcs_body(sp_lo, sp_hi, degp, odegp, slo_o, shi_o, cinv_o):
    slo_o[...] = sp_lo[0:N, :] + sp_lo[NPAD:NPAD + N, :]
    shi_o[...] = sp_hi[0:N, :] + sp_hi[NPAD:NPAD + N, :]
    cnt = (degp[0:N, 0:1] + degp[NPAD:NPAD + N, 0:1]
           + odegp[0:N, 0:1] + odegp[NPAD:NPAD + N, 0:1])
    cinv_o[...] = 1.0 / jnp.maximum(cnt, 1.0)


def _tc5_body(nr, s_lo_r, s_hi_r, cinv_r, ef,
              wa1, ba1, wa2, ba2, wp, bp, gp, bpln,
              wia, wib_lo, wib_hi, wic, bi, gi, biln,
              wq1a, wq1b, bq1, wq2, bq2,
              pred_o, ee_o, ie_o):
    nrv = nr[...]
    cinv = cinv_r[...]
    eagg_lo = nrv[:, 0:64] * s_lo_r[...] * cinv
    eagg_hi = nrv[:, 64:128] * s_hi_r[...] * cinv

    e = jnp.maximum(_mm(ef[...], wa1[...]) + ba1[...], 0.0)
    e = _mm(e, wa2[...]) + ba2[...]
    ee = jnp.maximum(_ln(_mm(e, wp[...]) + bp[...], gp[...], bpln[...]), 0.0)

    z = jnp.maximum(
        _mm(nrv, wia[...]) + _mm(eagg_lo, wib_lo[...])
        + _mm(eagg_hi, wib_hi[...]) + _mm(ee, wic[...]) + bi[...],
        0.0)
    ie = _ln(z, gi[...], biln[...])

    q = jnp.maximum(_mm(ee, wq1a[...]) + _mm(ie, wq1b[...]) + bq1[...], 0.0)
    logit = jnp.sum(q * wq2[...], axis=-1, keepdims=True) + bq2[...]
    pred_o[...] = jax.nn.sigmoid(logit)
    ee_o[...] = ee
    ie_o[...] = ie


def _tc_call(body, out_shapes):
    return pl.pallas_call(
        body, out_shape=[jax.ShapeDtypeStruct(s, _f32) for s in out_shapes])


B5 = 2000


def _tc5_call(out_shapes):
    def spec(shape):
        if shape[0] == N:
            return pl.BlockSpec((B5, shape[1]), lambda i: (i, 0))
        return pl.BlockSpec(shape, lambda i: (0, 0))

    def wrap(*arrays):
        in_specs = [spec(a.shape) for a in arrays]
        return pl.pallas_call(
            _tc5_body,
            grid=(N // B5,),
            in_specs=in_specs,
            out_specs=[pl.BlockSpec((B5, s[1]), lambda i: (i, 0))
                       for s in out_shapes],
            out_shape=[jax.ShapeDtypeStruct(s, _f32) for s in out_shapes],
        )(*arrays)
    return wrap


def kernel(x, edge_index, explicit_features, params):
    p = params
    src = edge_index[0].astype(jnp.int32)
    dst = edge_index[1].astype(jnp.int32)
    pad = jnp.full((E_PAD - E,), DUMMY, jnp.int32)
    src_i = jnp.concatenate([src, pad]).reshape(R_ALLOC, LANES)
    dst_i = jnp.concatenate([dst, pad]).reshape(R_ALLOC, LANES)
    zrows64 = jnp.zeros((RPT, 64), _f32)
    z16 = jnp.zeros((RPT, 16), _f32)
    ones16 = jnp.ones((LANES, 16), _f32)

    row = lambda v: v.reshape(1, -1)

    u1, r1 = _tc_call(_tc1_body, [(N16, 64), (N, 64)])(
        x, p['Wl1'], p['Wr1'], row(p['bl1']))
    acc1, deg_p, odeg_p = _make_seg_sum(64, True)(
        u1, src_i, dst_i, zrows64, z16, ones16)

    u2, r2, dinv = _tc_call(_tc2_body, [(N16, 64), (N, 64), (N, 1)])(
        acc1, deg_p, r1, p['Wl2'], p['Wr2'], row(p['bl2']))
    seg64 = _make_seg_sum(64, False)
    (acc2,) = seg64(u2, src_i, dst_i, zrows64)

    (h2,) = _tc_call(_tc3_body, [(N16, 64)])(acc2, dinv, r2)
    (acc3,) = seg64(h2, src_i, dst_i, zrows64)

    nr, nr_lo, nr_hi = _tc_call(_tc4_body, [(N, 128), (N16, 64), (N16, 64)])(
        acc3, dinv, h2, p['Wl3'], p['Wr3'], row(p['bl3']))

    bidir = _make_seg_sum_bidir(64)
    (s_p_lo,) = bidir(nr_lo, src_i, dst_i, zrows64)
    (s_p_hi,) = bidir(nr_hi, src_i, dst_i, zrows64)

    s_lo, s_hi, cinv = _tc_call(_tcs_body, [(N, 64), (N, 64), (N, 1)])(
        s_p_lo, s_p_hi, deg_p, odeg_p)

    wi = p['Wi']
    wq1 = p['Wq1']
    pred, ee, ie = _tc5_call([(N, 1), (N, 128), (N, 128)])(
        nr, s_lo, s_hi, cinv, explicit_features,
        p['Wa1'], row(p['ba1']), p['Wa2'], row(p['ba2']),
        p['Wp'], row(p['bp']), row(p['gp']), row(p['bp_ln']),
        wi[:, 0:128], wi[:, 128:192], wi[:, 192:256], wi[:, 256:384],
        row(p['bi']), row(p['gi']), row(p['bi_ln']),
        wq1[:, 0:128], wq1[:, 128:256], row(p['bq1']),
        p['Wq2'], row(p['bq2']))
    return pred, ee, ie


# speedup vs baseline: 4.3008x; 1.0382x over previous
import jax
import jax.numpy as jnp
from jax import lax
from jax.experimental import pallas as pl
from jax.experimental.pallas import tpu as pltpu
from jax.experimental.pallas import tpu_sc as plsc

N = 10000
N16 = 10016
E = 320000
NC, NS = 2, 16
LANES = 128
RT_BIG = 128
RT_SMALL = 32
BIG_CORE = 1
R_TOT = NS * (RT_BIG + RT_SMALL)
OFF_SMALL = NS * RT_BIG
R_ALLOC = R_TOT
E_PAD = R_ALLOC * LANES
NPAD = 10112
RPT = NPAD // NS
DUMMY = N

_f32 = jnp.float32


def _mesh():
    return plsc.VectorSubcoreMesh(
        core_axis_name="c", subcore_axis_name="s",
        num_cores=NC, num_subcores=NS)


def _sc_params():
    return pltpu.CompilerParams(use_tc_tiling_on_sc=False)


def _core_share(c, s):
    big = c == BIG_CORE
    rt = jnp.where(big, RT_BIG, RT_SMALL)
    base = jnp.where(big, s * RT_BIG, OFF_SMALL + s * RT_SMALL)
    return big, rt, base


def _stage_idx(big, base, hbm, vbuf):
    @pl.when(big)
    def _():
        pltpu.sync_copy(hbm.at[pl.ds(base, RT_BIG)], vbuf)

    @pl.when(jnp.logical_not(big))
    def _():
        pltpu.sync_copy(hbm.at[pl.ds(base, RT_SMALL)],
                        vbuf.at[pl.ds(0, RT_SMALL)])


def _wait_like(table, rows_slot, sem):
    pltpu.make_async_copy(table.at[pl.ds(0, LANES)], rows_slot, sem).wait()


def _seg_sum_body(count, table, gidx, sidx, zrows, *refs):
    if count:
        (z16, ones16, acc_out, deg_out, odeg_out,
         gv, sv, ones_v, rows, acc_sh, deg_sh, odeg_sh, *sems) = refs
        nslot, dist = 4, 2
        semg, sems_ = sems[:4], sems[4:8]
        sem_oi, sem_oo = sems[8], sems[9]
    else:
        (acc_out, gv, sv, rows, acc_sh, *sems) = refs
        nslot, dist = 6, 3
        semg, sems_ = sems[:6], sems[6:12]
    c = lax.axis_index("c")
    s = lax.axis_index("s")
    big, rt, base = _core_share(c, s)

    _stage_idx(big, base, gidx, gv)
    _stage_idx(big, base, sidx, sv)
    pltpu.sync_copy(zrows, acc_sh.at[pl.ds(s * RPT, RPT)])
    if count:
        pltpu.sync_copy(ones16, ones_v)
        pltpu.sync_copy(z16, deg_sh.at[pl.ds(s * RPT, RPT)])
        pltpu.sync_copy(z16, odeg_sh.at[pl.ds(s * RPT, RPT)])
    plsc.subcore_barrier()

    for b in range(dist):
        pltpu.async_copy(table.at[gv.at[b]], rows.at[b], semg[b])

    def emit(j, sl):
        sl_n = (sl + dist) % nslot
        _wait_like(table, rows.at[sl], semg[sl])
        pltpu.async_copy(rows.at[sl], acc_sh.at[sv.at[j]], sems_[sl],
                         add=True)
        if count:
            pltpu.async_copy(ones_v, deg_sh.at[sv.at[j]], sem_oi, add=True)
            pltpu.async_copy(ones_v, odeg_sh.at[gv.at[j]], sem_oo, add=True)

        @pl.when(j >= dist)
        def _():
            _wait_like(table, rows.at[sl_n], sems_[sl_n])
            if count:
                _wait_like(ones16, ones_v, sem_oi)
                _wait_like(ones16, ones_v, sem_oo)

        @pl.when(j + dist < rt)
        def _():
            pltpu.async_copy(table.at[gv.at[j + dist]], rows.at[sl_n],
                             semg[sl_n])

    if count:
        @pl.loop(0, rt, step=4)
        def _(j0):
            for b in range(4):
                emit(j0 + b, b)
    else:
        @pl.loop(0, rt - 2, step=6)
        def _(j0):
            for b in range(6):
                emit(j0 + b, b)
        emit(rt - 2, 0)
        emit(rt - 1, 1)

    drain = (2, 3) if count else (5, 0, 1)
    for sl in drain:
        _wait_like(table, rows.at[sl], sems_[sl])
    if count:
        for sem in (sem_oi, sem_oo):
            _wait_like(ones16, ones_v, sem)
            _wait_like(ones16, ones_v, sem)
    plsc.subcore_barrier()
    r0 = s * RPT
    o0 = c * NPAD + r0
    pltpu.sync_copy(acc_sh.at[pl.ds(r0, RPT)], acc_out.at[pl.ds(o0, RPT)])
    if count:
        pltpu.sync_copy(deg_sh.at[pl.ds(r0, RPT)], deg_out.at[pl.ds(o0, RPT)])
        pltpu.sync_copy(odeg_sh.at[pl.ds(r0, RPT)], odeg_out.at[pl.ds(o0, RPT)])


def _make_seg_sum(d, count):
    nslot = 4 if count else 6
    outs = [jax.ShapeDtypeStruct((NC * NPAD, d), _f32)]
    scratch = [
        pltpu.VMEM((RT_BIG, LANES), jnp.int32),
        pltpu.VMEM((RT_BIG, LANES), jnp.int32),
        pltpu.VMEM((nslot, LANES, d), _f32),
        pltpu.VMEM_SHARED((NPAD, d), _f32),
    ]
    nsem = 2 * nslot
    if count:
        outs += [jax.ShapeDtypeStruct((NC * NPAD, 16), _f32)] * 2
        scratch = scratch[:2] + [pltpu.VMEM((LANES, 16), _f32)] + scratch[2:]
        scratch += [pltpu.VMEM_SHARED((NPAD, 16), _f32)] * 2
        nsem += 2
    scratch += [pltpu.SemaphoreType.DMA] * nsem

    def body(*refs):
        _seg_sum_body(count, *refs)

    return pl.kernel(
        body, out_type=outs, mesh=_mesh(), scratch_types=scratch,
        compiler_params=_sc_params())


def _seg_sum_bidir_body(table, srci, dsti, zrows, acc_out,
                        sv_, dv_, rows, acc_sh, *sems):
    semg, sems_ = sems[:6], sems[6:12]
    c = lax.axis_index("c")
    s = lax.axis_index("s")
    big, rt, base = _core_share(c, s)
    _stage_idx(big, base, srci, sv_)
    _stage_idx(big, base, dsti, dv_)
    pltpu.sync_copy(zrows, acc_sh.at[pl.ds(s * RPT, RPT)])
    plsc.subcore_barrier()

    gref = (sv_, dv_)
    sref = (dv_, sv_)

    pltpu.async_copy(table.at[sv_.at[0]], rows.at[0], semg[0])
    pltpu.async_copy(table.at[dv_.at[0]], rows.at[1], semg[1])
    pltpu.async_copy(table.at[sv_.at[1]], rows.at[2], semg[2])

    def emit(j, sl, d):
        sl3 = (sl + 3) % 6
        _wait_like(table, rows.at[sl], semg[sl])
        pltpu.async_copy(rows.at[sl], acc_sh.at[sref[d].at[j]], sems_[sl],
                         add=True)

        @pl.when(2 * j + d >= 3)
        def _():
            _wait_like(table, rows.at[sl3], sems_[sl3])

        @pl.when(j + 1 + d < rt)
        def _():
            pltpu.async_copy(table.at[gref[1 - d].at[j + 1 + d]],
                             rows.at[sl3], semg[sl3])

    @pl.loop(0, rt - 2, step=3)
    def _(j0):
        for b in range(3):
            for d in range(2):
                emit(j0 + b, (2 * b + d) % 6, d)

    for b in range(2):
        for d in range(2):
            emit(rt - 2 + b, 2 * b + d, d)

    for sl in (1, 2, 3):
        _wait_like(table, rows.at[sl], sems_[sl])
    plsc.subcore_barrier()
    r0 = s * RPT
    pltpu.sync_copy(acc_sh.at[pl.ds(r0, RPT)],
                    acc_out.at[pl.ds(c * NPAD + r0, RPT)])


def _make_seg_sum_bidir(d):
    scratch = [pltpu.VMEM((RT_BIG, LANES), jnp.int32) for _ in range(2)] + [
        pltpu.VMEM((6, LANES, d), _f32),
        pltpu.VMEM_SHARED((NPAD, d), _f32),
    ] + [pltpu.SemaphoreType.DMA] * 12
    return pl.kernel(
        _seg_sum_bidir_body,
        out_type=[jax.ShapeDtypeStruct((NC * NPAD, d), _f32)],
        mesh=_mesh(), scratch_types=scratch,
        compiler_params=_sc_params())


def _mm(a, w):
    return lax.dot_general(a, w, (((1,), (1,)), ((), ())),
                           preferred_element_type=_f32)


def _ln(x, g, b, eps=1e-5):
    mu = jnp.mean(x, axis=-1, keepdims=True)
    var = jnp.mean((x - mu) ** 2, axis=-1, keepdims=True)
    return (x - mu) / jnp.sqrt(var + eps) * g + b


def _pad16(x):
    return jnp.concatenate([x, jnp.zeros((N16 - N, x.shape[1]), _f32)], 0)


def _tc1_body(x, wl1, wr1, bl1, u1_o, r1_o):
    xv = x[...]
    u1_o[...] = _pad16(_mm(xv, wl1[...]))
    r1_o[...] = _mm(xv, wr1[...]) + bl1[...]


def _tc2_body(acc1, degp, r1, wl2, wr2, bl2, u2_o, r2_o, dinv_o):
    a = acc1[0:N, :] + acc1[NPAD:NPAD + N, :]
    deg = degp[0:N, 0:1] + degp[NPAD:NPAD + N, 0:1]
    dinv = 1.0 / jnp.maximum(deg, 1.0)
    h = jnp.maximum(a * dinv + r1[...], 0.0)
    u2_o[...] = _pad16(_mm(h, wl2[...]))
    r2_o[...] = _mm(h, wr2[...]) + bl2[...]
    dinv_o[...] = dinv


def _tc3_body(acc2, dinv, r2, h2_o):
    a = acc2[0:N, :] + acc2[NPAD:NPAD + N, :]
    h2_o[...] = _pad16(jnp.maximum(a * dinv[...] + r2[...], 0.0))


def _tc4_body(acc3, dinv, h2, wl3, wr3, bl3, nr_o, nrlo_o, nrhi_o):
    h2v = h2[0:N, :]
    a = (acc3[0:N, :] + acc3[NPAD:NPAD + N, :]) * dinv[...]
    nr = _mm(a, wl3[...]) + bl3[...] + _mm(h2v, wr3[...])
    nr_o[...] = nr
    nrlo_o[...] = _pad16(nr[:, 0:64])
    nrhi_o[...] = _pad16(nr[:, 64:128])


def _tcs_body(sp_lo, sp_hi, degp, odegp, slo_o, shi_o, cinv_o):
    slo_o[...] = sp_lo[0:N, :] + sp_lo[NPAD:NPAD + N, :]
    shi_o[...] = sp_hi[0:N, :] + sp_hi[NPAD:NPAD + N, :]
    cnt = (degp[0:N, 0:1] + degp[NPAD:NPAD + N, 0:1]
           + odegp[0:N, 0:1] + odegp[NPAD:NPAD + N, 0:1])
    cinv_o[...] = 1.0 / jnp.maximum(cnt, 1.0)


def _tc5_body(nr, s_lo_r, s_hi_r, cinv_r, ef,
              wa1, ba1, wa2, ba2, wp, bp, gp, bpln,
              wia, wib_lo, wib_hi, wic, bi, gi, biln,
              wq1a, wq1b, bq1, wq2, bq2,
              pred_o, ee_o, ie_o):
    nrv = nr[...]
    cinv = cinv_r[...]
    eagg_lo = nrv[:, 0:64] * s_lo_r[...] * cinv
    eagg_hi = nrv[:, 64:128] * s_hi_r[...] * cinv

    e = jnp.maximum(_mm(ef[...], wa1[...]) + ba1[...], 0.0)
    e = _mm(e, wa2[...]) + ba2[...]
    ee = jnp.maximum(_ln(_mm(e, wp[...]) + bp[...], gp[...], bpln[...]), 0.0)

    z = jnp.maximum(
        _mm(nrv, wia[...]) + _mm(eagg_lo, wib_lo[...])
        + _mm(eagg_hi, wib_hi[...]) + _mm(ee, wic[...]) + bi[...],
        0.0)
    ie = _ln(z, gi[...], biln[...])

    q = jnp.maximum(_mm(ee, wq1a[...]) + _mm(ie, wq1b[...]) + bq1[...], 0.0)
    logit = jnp.sum(q * wq2[...], axis=-1, keepdims=True) + bq2[...]
    pred_o[...] = jax.nn.sigmoid(logit)
    ee_o[...] = ee
    ie_o[...] = ie


def _tc_call(body, out_shapes):
    return pl.pallas_call(
        body, out_shape=[jax.ShapeDtypeStruct(s, _f32) for s in out_shapes])


B5 = 2000


def _tc5_call(out_shapes):
    def spec(shape):
        if shape[0] == N:
            return pl.BlockSpec((B5, shape[1]), lambda i: (i, 0))
        return pl.BlockSpec(shape, lambda i: (0, 0))

    def wrap(*arrays):
        in_specs = [spec(a.shape) for a in arrays]
        return pl.pallas_call(
            _tc5_body,
            grid=(N // B5,),
            in_specs=in_specs,
            out_specs=[pl.BlockSpec((B5, s[1]), lambda i: (i, 0))
                       for s in out_shapes],
            out_shape=[jax.ShapeDtypeStruct(s, _f32) for s in out_shapes],
        )(*arrays)
    return wrap


def kernel(x, edge_index, explicit_features, params):
    p = params
    src = edge_index[0].astype(jnp.int32)
    dst = edge_index[1].astype(jnp.int32)
    pad = jnp.full((E_PAD - E,), DUMMY, jnp.int32)
    src_i = jnp.concatenate([src, pad]).reshape(R_ALLOC, LANES)
    dst_i = jnp.concatenate([dst, pad]).reshape(R_ALLOC, LANES)
    zrows64 = jnp.zeros((RPT, 64), _f32)
    z16 = jnp.zeros((RPT, 16), _f32)
    ones16 = jnp.ones((LANES, 16), _f32)

    row = lambda v: v.reshape(1, -1)

    u1, r1 = _tc_call(_tc1_body, [(N16, 64), (N, 64)])(
        x, p['Wl1'], p['Wr1'], row(p['bl1']))
    acc1, deg_p, odeg_p = _make_seg_sum(64, True)(
        u1, src_i, dst_i, zrows64, z16, ones16)

    u2, r2, dinv = _tc_call(_tc2_body, [(N16, 64), (N, 64), (N, 1)])(
        acc1, deg_p, r1, p['Wl2'], p['Wr2'], row(p['bl2']))
    seg64 = _make_seg_sum(64, False)
    (acc2,) = seg64(u2, src_i, dst_i, zrows64)

    (h2,) = _tc_call(_tc3_body, [(N16, 64)])(acc2, dinv, r2)
    (acc3,) = seg64(h2, src_i, dst_i, zrows64)

    nr, nr_lo, nr_hi = _tc_call(_tc4_body, [(N, 128), (N16, 64), (N16, 64)])(
        acc3, dinv, h2, p['Wl3'], p['Wr3'], row(p['bl3']))

    bidir = _make_seg_sum_bidir(64)
    (s_p_lo,) = bidir(nr_lo, src_i, dst_i, zrows64)
    (s_p_hi,) = bidir(nr_hi, src_i, dst_i, zrows64)

    s_lo, s_hi, cinv = _tc_call(_tcs_body, [(N, 64), (N, 64), (N, 1)])(
        s_p_lo, s_p_hi, deg_p, odeg_p)

    wi = p['Wi']
    wq1 = p['Wq1']
    pred, ee, ie = _tc5_call([(N, 1), (N, 128), (N, 128)])(
        nr, s_lo, s_hi, cinv, explicit_features,
        p['Wa1'], row(p['ba1']), p['Wa2'], row(p['ba2']),
        p['Wp'], row(p['bp']), row(p['gp']), row(p['bp_ln']),
        wi[:, 0:128], wi[:, 128:192], wi[:, 192:256], wi[:, 256:384],
        row(p['bi']), row(p['gi']), row(p['bi_ln']),
        wq1[:, 0:128], wq1[:, 128:256], row(p['bq1']),
        p['Wq2'], row(p['bq2']))
    return pred, ee, ie


# speedup vs baseline: 4.3313x; 1.0071x over previous
import jax
import jax.numpy as jnp
from jax import lax
from jax.experimental import pallas as pl
from jax.experimental.pallas import tpu as pltpu
from jax.experimental.pallas import tpu_sc as plsc

N = 10000
N16 = 10016
E = 320000
NC, NS = 2, 16
LANES = 128
RT_BIG = 128
RT_SMALL = 32
BIG_CORE = 1
R_TOT = NS * (RT_BIG + RT_SMALL)
OFF_SMALL = NS * RT_BIG
R_ALLOC = R_TOT
E_PAD = R_ALLOC * LANES
NPAD = 10112
RPT = NPAD // NS
DUMMY = N

_f32 = jnp.float32


def _mesh():
    return plsc.VectorSubcoreMesh(
        core_axis_name="c", subcore_axis_name="s",
        num_cores=NC, num_subcores=NS)


def _sc_params():
    return pltpu.CompilerParams(use_tc_tiling_on_sc=False)


def _core_share(c, s):
    big = c == BIG_CORE
    rt = jnp.where(big, RT_BIG, RT_SMALL)
    base = jnp.where(big, s * RT_BIG, OFF_SMALL + s * RT_SMALL)
    return big, rt, base


def _stage_idx(big, base, hbm, vbuf):
    @pl.when(big)
    def _():
        pltpu.sync_copy(hbm.at[pl.ds(base, RT_BIG)], vbuf)

    @pl.when(jnp.logical_not(big))
    def _():
        pltpu.sync_copy(hbm.at[pl.ds(base, RT_SMALL)],
                        vbuf.at[pl.ds(0, RT_SMALL)])


def _wait_like(table, rows_slot, sem):
    pltpu.make_async_copy(table.at[pl.ds(0, LANES)], rows_slot, sem).wait()


def _seg_sum_body(count, table, gidx, sidx, zrows, *refs):
    if count:
        (z16, ones16, acc_out, deg_out, odeg_out,
         gv, sv, ones_v, rows, acc_sh, deg_sh, odeg_sh, *sems) = refs
        nslot, dist = 4, 2
        semg, sems_ = sems[:4], sems[4:8]
        sem_oi, sem_oo = sems[8], sems[9]
    else:
        (acc_out, gv, sv, rows, acc_sh, *sems) = refs
        nslot, dist = 6, 3
        semg, sems_ = sems[:6], sems[6:12]
    c = lax.axis_index("c")
    s = lax.axis_index("s")
    big, rt, base = _core_share(c, s)

    _stage_idx(big, base, gidx, gv)
    _stage_idx(big, base, sidx, sv)
    pltpu.sync_copy(zrows, acc_sh.at[pl.ds(s * RPT, RPT)])
    if count:
        pltpu.sync_copy(ones16, ones_v)
        pltpu.sync_copy(z16, deg_sh.at[pl.ds(s * RPT, RPT)])
        pltpu.sync_copy(z16, odeg_sh.at[pl.ds(s * RPT, RPT)])
    plsc.subcore_barrier()

    for b in range(dist):
        pltpu.async_copy(table.at[gv.at[b]], rows.at[b], semg[b])

    def emit(j, sl):
        sl_n = (sl + dist) % nslot
        _wait_like(table, rows.at[sl], semg[sl])
        pltpu.async_copy(rows.at[sl], acc_sh.at[sv.at[j]], sems_[sl],
                         add=True)
        if count:
            pltpu.async_copy(ones_v, deg_sh.at[sv.at[j]], sem_oi, add=True)
            pltpu.async_copy(ones_v, odeg_sh.at[gv.at[j]], sem_oo, add=True)

        @pl.when(j >= dist)
        def _():
            _wait_like(table, rows.at[sl_n], sems_[sl_n])
            if count:
                _wait_like(ones16, ones_v, sem_oi)
                _wait_like(ones16, ones_v, sem_oo)

        @pl.when(j + dist < rt)
        def _():
            pltpu.async_copy(table.at[gv.at[j + dist]], rows.at[sl_n],
                             semg[sl_n])

    if count:
        @pl.loop(0, rt, step=4)
        def _(j0):
            for b in range(4):
                emit(j0 + b, b)
    else:
        @pl.loop(0, rt - 2, step=6)
        def _(j0):
            for b in range(6):
                emit(j0 + b, b)
        emit(rt - 2, 0)
        emit(rt - 1, 1)

    drain = (2, 3) if count else (5, 0, 1)
    for sl in drain:
        _wait_like(table, rows.at[sl], sems_[sl])
    if count:
        for sem in (sem_oi, sem_oo):
            _wait_like(ones16, ones_v, sem)
            _wait_like(ones16, ones_v, sem)
    plsc.subcore_barrier()
    r0 = s * RPT
    o0 = c * NPAD + r0
    pltpu.sync_copy(acc_sh.at[pl.ds(r0, RPT)], acc_out.at[pl.ds(o0, RPT)])
    if count:
        pltpu.sync_copy(deg_sh.at[pl.ds(r0, RPT)], deg_out.at[pl.ds(o0, RPT)])
        pltpu.sync_copy(odeg_sh.at[pl.ds(r0, RPT)], odeg_out.at[pl.ds(o0, RPT)])


def _make_seg_sum(d, count):
    nslot = 4 if count else 6
    outs = [jax.ShapeDtypeStruct((NC * NPAD, d), _f32)]
    scratch = [
        pltpu.VMEM((RT_BIG, LANES), jnp.int32),
        pltpu.VMEM((RT_BIG, LANES), jnp.int32),
        pltpu.VMEM((nslot, LANES, d), _f32),
        pltpu.VMEM_SHARED((NPAD, d), _f32),
    ]
    nsem = 2 * nslot
    if count:
        outs += [jax.ShapeDtypeStruct((NC * NPAD, 16), _f32)] * 2
        scratch = scratch[:2] + [pltpu.VMEM((LANES, 16), _f32)] + scratch[2:]
        scratch += [pltpu.VMEM_SHARED((NPAD, 16), _f32)] * 2
        nsem += 2
    scratch += [pltpu.SemaphoreType.DMA] * nsem

    def body(*refs):
        _seg_sum_body(count, *refs)

    return pl.kernel(
        body, out_type=outs, mesh=_mesh(), scratch_types=scratch,
        compiler_params=_sc_params())


def _seg_sum_bidir_body(tlo, thi, srci, dsti, zrows, acc_out,
                        sv_, dv_, rows, acc_sh, *sems):
    semg, sems_ = sems[:6], sems[6:12]
    c = lax.axis_index("c")
    s = lax.axis_index("s")
    big, rt, base = _core_share(c, s)
    _stage_idx(big, base, srci, sv_)
    _stage_idx(big, base, dsti, dv_)

    gref = (sv_, dv_)
    sref = (dv_, sv_)
    r0 = s * RPT

    def phase(table, out_base):
        pltpu.sync_copy(zrows, acc_sh.at[pl.ds(r0, RPT)])
        plsc.subcore_barrier()

        pltpu.async_copy(table.at[sv_.at[0]], rows.at[0], semg[0])
        pltpu.async_copy(table.at[dv_.at[0]], rows.at[1], semg[1])
        pltpu.async_copy(table.at[sv_.at[1]], rows.at[2], semg[2])

        def emit(j, sl, d):
            sl3 = (sl + 3) % 6
            _wait_like(table, rows.at[sl], semg[sl])
            pltpu.async_copy(rows.at[sl], acc_sh.at[sref[d].at[j]],
                             sems_[sl], add=True)

            @pl.when(2 * j + d >= 3)
            def _():
                _wait_like(table, rows.at[sl3], sems_[sl3])

            @pl.when(j + 1 + d < rt)
            def _():
                pltpu.async_copy(table.at[gref[1 - d].at[j + 1 + d]],
                                 rows.at[sl3], semg[sl3])

        @pl.loop(0, rt - 2, step=3)
        def _(j0):
            for b in range(3):
                for d in range(2):
                    emit(j0 + b, (2 * b + d) % 6, d)

        for b in range(2):
            for d in range(2):
                emit(rt - 2 + b, 2 * b + d, d)

        for sl in (1, 2, 3):
            _wait_like(table, rows.at[sl], sems_[sl])
        plsc.subcore_barrier()
        pltpu.sync_copy(acc_sh.at[pl.ds(r0, RPT)],
                        acc_out.at[pl.ds(out_base + r0, RPT)])

    phase(tlo, (c * 2 + 0) * NPAD)
    phase(thi, (c * 2 + 1) * NPAD)


def _make_seg_sum_bidir(d):
    scratch = [pltpu.VMEM((RT_BIG, LANES), jnp.int32) for _ in range(2)] + [
        pltpu.VMEM((6, LANES, d), _f32),
        pltpu.VMEM_SHARED((NPAD, d), _f32),
    ] + [pltpu.SemaphoreType.DMA] * 12
    return pl.kernel(
        _seg_sum_bidir_body,
        out_type=[jax.ShapeDtypeStruct((NC * 2 * NPAD, d), _f32)],
        mesh=_mesh(), scratch_types=scratch,
        compiler_params=_sc_params())


def _mm(a, w):
    return lax.dot_general(a, w, (((1,), (1,)), ((), ())),
                           preferred_element_type=_f32)


def _ln(x, g, b, eps=1e-5):
    mu = jnp.mean(x, axis=-1, keepdims=True)
    var = jnp.mean((x - mu) ** 2, axis=-1, keepdims=True)
    return (x - mu) / jnp.sqrt(var + eps) * g + b


def _pad16(x):
    return jnp.concatenate([x, jnp.zeros((N16 - N, x.shape[1]), _f32)], 0)


def _tc1_body(x, wl1, wr1, bl1, u1_o, r1_o):
    xv = x[...]
    u1_o[...] = _pad16(_mm(xv, wl1[...]))
    r1_o[...] = _mm(xv, wr1[...]) + bl1[...]


def _tc2_body(acc1, degp, r1, wl2, wr2, bl2, u2_o, r2_o, dinv_o):
    a = acc1[0:N, :] + acc1[NPAD:NPAD + N, :]
    deg = degp[0:N, 0:1] + degp[NPAD:NPAD + N, 0:1]
    dinv = 1.0 / jnp.maximum(deg, 1.0)
    h = jnp.maximum(a * dinv + r1[...], 0.0)
    u2_o[...] = _pad16(_mm(h, wl2[...]))
    r2_o[...] = _mm(h, wr2[...]) + bl2[...]
    dinv_o[...] = dinv


def _tc3_body(acc2, dinv, r2, h2_o):
    a = acc2[0:N, :] + acc2[NPAD:NPAD + N, :]
    h2_o[...] = _pad16(jnp.maximum(a * dinv[...] + r2[...], 0.0))


def _tc4_body(acc3, dinv, h2, wl3, wr3, bl3, nr_o, nrlo_o, nrhi_o):
    h2v = h2[0:N, :]
    a = (acc3[0:N, :] + acc3[NPAD:NPAD + N, :]) * dinv[...]
    nr = _mm(a, wl3[...]) + bl3[...] + _mm(h2v, wr3[...])
    nr_o[...] = nr
    nrlo_o[...] = _pad16(nr[:, 0:64])
    nrhi_o[...] = _pad16(nr[:, 64:128])


def _tcs_body(sp, degp, odegp, slo_o, shi_o, cinv_o):
    slo_o[...] = sp[0:N, :] + sp[2 * NPAD:2 * NPAD + N, :]
    shi_o[...] = sp[NPAD:NPAD + N, :] + sp[3 * NPAD:3 * NPAD + N, :]
    cnt = (degp[0:N, 0:1] + degp[NPAD:NPAD + N, 0:1]
           + odegp[0:N, 0:1] + odegp[NPAD:NPAD + N, 0:1])
    cinv_o[...] = 1.0 / jnp.maximum(cnt, 1.0)


def _tc5_body(nr, s_lo_r, s_hi_r, cinv_r, ef,
              wa1, ba1, wa2, ba2, wp, bp, gp, bpln,
              wia, wib_lo, wib_hi, wic, bi, gi, biln,
              wq1a, wq1b, bq1, wq2, bq2,
              pred_o, ee_o, ie_o):
    nrv = nr[...]
    cinv = cinv_r[...]
    eagg_lo = nrv[:, 0:64] * s_lo_r[...] * cinv
    eagg_hi = nrv[:, 64:128] * s_hi_r[...] * cinv

    e = jnp.maximum(_mm(ef[...], wa1[...]) + ba1[...], 0.0)
    e = _mm(e, wa2[...]) + ba2[...]
    ee = jnp.maximum(_ln(_mm(e, wp[...]) + bp[...], gp[...], bpln[...]), 0.0)

    z = jnp.maximum(
        _mm(nrv, wia[...]) + _mm(eagg_lo, wib_lo[...])
        + _mm(eagg_hi, wib_hi[...]) + _mm(ee, wic[...]) + bi[...],
        0.0)
    ie = _ln(z, gi[...], biln[...])

    q = jnp.maximum(_mm(ee, wq1a[...]) + _mm(ie, wq1b[...]) + bq1[...], 0.0)
    logit = jnp.sum(q * wq2[...], axis=-1, keepdims=True) + bq2[...]
    pred_o[...] = jax.nn.sigmoid(logit)
    ee_o[...] = ee
    ie_o[...] = ie


def _tc_call(body, out_shapes):
    return pl.pallas_call(
        body, out_shape=[jax.ShapeDtypeStruct(s, _f32) for s in out_shapes])


B5 = 2000


def _tc5_call(out_shapes):
    def spec(shape):
        if shape[0] == N:
            return pl.BlockSpec((B5, shape[1]), lambda i: (i, 0))
        return pl.BlockSpec(shape, lambda i: (0, 0))

    def wrap(*arrays):
        in_specs = [spec(a.shape) for a in arrays]
        return pl.pallas_call(
            _tc5_body,
            grid=(N // B5,),
            in_specs=in_specs,
            out_specs=[pl.BlockSpec((B5, s[1]), lambda i: (i, 0))
                       for s in out_shapes],
            out_shape=[jax.ShapeDtypeStruct(s, _f32) for s in out_shapes],
        )(*arrays)
    return wrap


def kernel(x, edge_index, explicit_features, params):
    p = params
    src = edge_index[0].astype(jnp.int32)
    dst = edge_index[1].astype(jnp.int32)
    pad = jnp.full((E_PAD - E,), DUMMY, jnp.int32)
    src_i = jnp.concatenate([src, pad]).reshape(R_ALLOC, LANES)
    dst_i = jnp.concatenate([dst, pad]).reshape(R_ALLOC, LANES)
    zrows64 = jnp.zeros((RPT, 64), _f32)
    z16 = jnp.zeros((RPT, 16), _f32)
    ones16 = jnp.ones((LANES, 16), _f32)

    row = lambda v: v.reshape(1, -1)

    u1, r1 = _tc_call(_tc1_body, [(N16, 64), (N, 64)])(
        x, p['Wl1'], p['Wr1'], row(p['bl1']))
    acc1, deg_p, odeg_p = _make_seg_sum(64, True)(
        u1, src_i, dst_i, zrows64, z16, ones16)

    u2, r2, dinv = _tc_call(_tc2_body, [(N16, 64), (N, 64), (N, 1)])(
        acc1, deg_p, r1, p['Wl2'], p['Wr2'], row(p['bl2']))
    seg64 = _make_seg_sum(64, False)
    (acc2,) = seg64(u2, src_i, dst_i, zrows64)

    (h2,) = _tc_call(_tc3_body, [(N16, 64)])(acc2, dinv, r2)
    (acc3,) = seg64(h2, src_i, dst_i, zrows64)

    nr, nr_lo, nr_hi = _tc_call(_tc4_body, [(N, 128), (N16, 64), (N16, 64)])(
        acc3, dinv, h2, p['Wl3'], p['Wr3'], row(p['bl3']))

    (s_p,) = _make_seg_sum_bidir(64)(nr_lo, nr_hi, src_i, dst_i, zrows64)

    s_lo, s_hi, cinv = _tc_call(_tcs_body, [(N, 64), (N, 64), (N, 1)])(
        s_p, deg_p, odeg_p)

    wi = p['Wi']
    wq1 = p['Wq1']
    pred, ee, ie = _tc5_call([(N, 1), (N, 128), (N, 128)])(
        nr, s_lo, s_hi, cinv, explicit_features,
        p['Wa1'], row(p['ba1']), p['Wa2'], row(p['ba2']),
        p['Wp'], row(p['bp']), row(p['gp']), row(p['bp_ln']),
        wi[:, 0:128], wi[:, 128:192], wi[:, 192:256], wi[:, 256:384],
        row(p['bi']), row(p['gi']), row(p['bi_ln']),
        wq1[:, 0:128], wq1[:, 128:256], row(p['bq1']),
        p['Wq2'], row(p['bq2']))
    return pred, ee, ie
